# baseline (speedup 1.0000x reference)
import functools

import jax
import jax.numpy as jnp
from jax import lax
from jax.experimental import pallas as pl
from jax.experimental.pallas import tpu as pltpu
from jax.experimental.pallas import tpu_sc as plsc

N = 10000
DEG = 16
D = 256
HIDDEN = 512
OUT = 256
BN = 400
A = 5600
SCN = N - A
CB = 8
LANES = 16


def _sc_edge_sum(x3d):
    mesh = plsc.VectorSubcoreMesh(core_axis_name="c", subcore_axis_name="s")
    chunk_off = A // CB

    @functools.partial(
        pl.kernel,
        out_type=jax.ShapeDtypeStruct((SCN, D), jnp.float32),
        mesh=mesh,
    )
    def k(x_hbm, o_hbm):
        def body(in_ref, out_ref):
            @pl.loop(0, CB)
            def _(j):
                for c in range(0, D, LANES):
                    acc = in_ref.at[j, 0, pl.ds(c, LANES)][...]
                    for kk in range(1, DEG):
                        acc = acc + in_ref.at[j, kk, pl.ds(c, LANES)][...]
                    out_ref.at[j, pl.ds(c, LANES)][...] = acc

        pltpu.emit_pipeline(
            body,
            grid=(SCN // CB,),
            in_specs=[pl.BlockSpec((CB, DEG, D), lambda i: (i + chunk_off, 0, 0))],
            out_specs=[pl.BlockSpec((CB, D), lambda i: (i, 0))],
            core_axis_name=("c", "s"),
            dimension_semantics=(pltpu.PARALLEL,),
        )(x_hbm, o_hbm)

    return k(x3d)


def _mlp(x_e, node_msg, nh, nf, w1_ref, b1_ref, w2_ref, b2_ref, w3_ref, b3_ref):
    h = (jnp.dot(x_e, w1_ref[0 * D:1 * D], preferred_element_type=jnp.float32)
         + jnp.dot(node_msg, w1_ref[1 * D:2 * D], preferred_element_type=jnp.float32)
         + jnp.dot(nh, w1_ref[2 * D:3 * D], preferred_element_type=jnp.float32)
         + jnp.dot(nf, w1_ref[3 * D:4 * D], preferred_element_type=jnp.float32)
         + b1_ref[...])
    h = jnp.maximum(h, 0.0)
    h = jnp.dot(h, w2_ref[...], preferred_element_type=jnp.float32) + b2_ref[...]
    h = jnp.maximum(h, 0.0)
    return jnp.dot(h, w3_ref[...], preferred_element_type=jnp.float32) + b3_ref[...]


def _fused_full(mnh_ref, attn_ref, meh_ref, nh_ref, nf_ref,
                w1_ref, b1_ref, w2_ref, b2_ref, w3_ref, b3_ref, out_ref):
    attn = attn_ref[...]
    node_msg = jnp.sum(mnh_ref[...] * attn[:, :, None], axis=1)
    edge_msg = jnp.sum(meh_ref[...], axis=1)
    out_ref[...] = _mlp(edge_msg, node_msg, nh_ref[...], nf_ref[...],
                        w1_ref, b1_ref, w2_ref, b2_ref, w3_ref, b3_ref)


def _fused_consume(mnh_ref, attn_ref, em_ref, nh_ref, nf_ref,
                   w1_ref, b1_ref, w2_ref, b2_ref, w3_ref, b3_ref,
                   head_ref, out_ref):
    attn = attn_ref[...]
    node_msg = jnp.sum(mnh_ref[...] * attn[:, :, None], axis=1)
    out_ref[...] = _mlp(em_ref[...], node_msg, nh_ref[...], nf_ref[...],
                        w1_ref, b1_ref, w2_ref, b2_ref, w3_ref, b3_ref)


def _weight_specs():
    fixed = lambda i: (0, 0)
    return [
        pl.BlockSpec((4 * D, HIDDEN), fixed),
        pl.BlockSpec((1, HIDDEN), fixed),
        pl.BlockSpec((HIDDEN, HIDDEN), fixed),
        pl.BlockSpec((1, HIDDEN), fixed),
        pl.BlockSpec((HIDDEN, OUT), fixed),
        pl.BlockSpec((1, OUT), fixed),
    ]


def kernel(mailbox_node_h, mailbox_attn, mailbox_edge_h, node_h, node_features,
           W1, b1, W2, b2, W3, b3):
    attn2d = mailbox_attn[:, :, 0]
    b1r = b1.reshape(1, HIDDEN)
    b2r = b2.reshape(1, HIDDEN)
    b3r = b3.reshape(1, OUT)
    weights = (W1, b1r, W2, b2r, W3, b3r)

    em_tail = _sc_edge_sum(mailbox_edge_h)

    row = lambda off: (lambda i: (i + off, 0))
    row3 = lambda off: (lambda i: (i + off, 0, 0))

    out_head = pl.pallas_call(
        _fused_full,
        grid=(A // BN,),
        in_specs=[
            pl.BlockSpec((BN, DEG, D), row3(0)),
            pl.BlockSpec((BN, DEG), row(0)),
            pl.BlockSpec((BN, DEG, D), row3(0)),
            pl.BlockSpec((BN, D), row(0)),
            pl.BlockSpec((BN, D), row(0)),
        ] + _weight_specs(),
        out_specs=pl.BlockSpec((BN, OUT), row(0)),
        out_shape=jax.ShapeDtypeStruct((N, OUT), jnp.float32),
        compiler_params=pltpu.CompilerParams(
            dimension_semantics=("arbitrary",),
        ),
    )(mailbox_node_h, attn2d, mailbox_edge_h, node_h, node_features,
      *weights)

    off = A // BN
    return pl.pallas_call(
        _fused_consume,
        grid=((N - A) // BN,),
        in_specs=[
            pl.BlockSpec((BN, DEG, D), row3(off)),
            pl.BlockSpec((BN, DEG), row(off)),
            pl.BlockSpec((BN, D), row(0)),
            pl.BlockSpec((BN, D), row(off)),
            pl.BlockSpec((BN, D), row(off)),
        ] + _weight_specs() + [
            pl.BlockSpec(memory_space=pltpu.MemorySpace.HBM),
        ],
        out_specs=pl.BlockSpec((BN, OUT), row(off)),
        out_shape=jax.ShapeDtypeStruct((N, OUT), jnp.float32),
        input_output_aliases={11: 0},
        compiler_params=pltpu.CompilerParams(
            dimension_semantics=("arbitrary",),
        ),
    )(mailbox_node_h, attn2d, em_tail, node_h, node_features,
      *weights, out_head)


# speedup vs baseline: 1.0908x; 1.0908x over previous
import jax
import jax.numpy as jnp
from jax.experimental import pallas as pl
from jax.experimental.pallas import tpu as pltpu

N = 10000
DEG = 16
D = 256
HIDDEN = 512
OUT = 256
BN = 400
H = DEG // 2


def _mlp(x_e, node_msg, nh, nf, w1_ref, b1_ref, w2_ref, b2_ref, w3_ref, b3_ref):
    h = (jnp.dot(x_e, w1_ref[0 * D:1 * D], preferred_element_type=jnp.float32)
         + jnp.dot(node_msg, w1_ref[1 * D:2 * D], preferred_element_type=jnp.float32)
         + jnp.dot(nh, w1_ref[2 * D:3 * D], preferred_element_type=jnp.float32)
         + jnp.dot(nf, w1_ref[3 * D:4 * D], preferred_element_type=jnp.float32)
         + b1_ref[...])
    h = jnp.maximum(h, 0.0)
    h = jnp.dot(h, w2_ref[...], preferred_element_type=jnp.float32) + b2_ref[...]
    h = jnp.maximum(h, 0.0)
    return jnp.dot(h, w3_ref[...], preferred_element_type=jnp.float32) + b3_ref[...]


def _fused(mnh_a_ref, mnh_b_ref, attn_ref, meh_a_ref, meh_b_ref, nh_ref, nf_ref,
           w1_ref, b1_ref, w2_ref, b2_ref, w3_ref, b3_ref, out_ref):
    attn = attn_ref[...]
    node_msg = (jnp.sum(mnh_a_ref[...] * attn[:, :H, None], axis=1)
                + jnp.sum(mnh_b_ref[...] * attn[:, H:, None], axis=1))
    edge_msg = jnp.sum(meh_a_ref[...], axis=1) + jnp.sum(meh_b_ref[...], axis=1)
    out_ref[...] = _mlp(edge_msg, node_msg, nh_ref[...], nf_ref[...],
                        w1_ref, b1_ref, w2_ref, b2_ref, w3_ref, b3_ref)


def kernel(mailbox_node_h, mailbox_attn, mailbox_edge_h, node_h, node_features,
           W1, b1, W2, b2, W3, b3):
    attn2d = mailbox_attn[:, :, 0]
    b1r = b1.reshape(1, HIDDEN)
    b2r = b2.reshape(1, HIDDEN)
    b3r = b3.reshape(1, OUT)

    grid = (N // BN,)
    row = lambda i: (i, 0)
    lo3 = lambda i: (i, 0, 0)
    hi3 = lambda i: (i, 1, 0)
    fixed = lambda i: (0, 0)

    return pl.pallas_call(
        _fused,
        grid=grid,
        in_specs=[
            pl.BlockSpec((BN, H, D), lo3),
            pl.BlockSpec((BN, H, D), hi3),
            pl.BlockSpec((BN, DEG), row),
            pl.BlockSpec((BN, H, D), lo3),
            pl.BlockSpec((BN, H, D), hi3),
            pl.BlockSpec((BN, D), row),
            pl.BlockSpec((BN, D), row),
            pl.BlockSpec((4 * D, HIDDEN), fixed),
            pl.BlockSpec((1, HIDDEN), fixed),
            pl.BlockSpec((HIDDEN, HIDDEN), fixed),
            pl.BlockSpec((1, HIDDEN), fixed),
            pl.BlockSpec((HIDDEN, OUT), fixed),
            pl.BlockSpec((1, OUT), fixed),
        ],
        out_specs=pl.BlockSpec((BN, OUT), row),
        out_shape=jax.ShapeDtypeStruct((N, OUT), jnp.float32),
        compiler_params=pltpu.CompilerParams(
            dimension_semantics=("arbitrary",),
        ),
    )(mailbox_node_h, mailbox_node_h, attn2d, mailbox_edge_h, mailbox_edge_h,
      node_h, node_features, W1, b1r, W2, b2r, W3, b3r)


# speedup vs baseline: 1.2184x; 1.1169x over previous
import jax
import jax.numpy as jnp
from jax.experimental import pallas as pl
from jax.experimental.pallas import tpu as pltpu

N = 10000
DEG = 16
D = 256
HIDDEN = 512
OUT = 256
BN = 400


def _fused(mnh_ref, attn_ref, meh_ref, nh_ref, nf_ref,
           w1_ref, b1_ref, w2_ref, b2_ref, w3_ref, b3_ref, out_ref):
    attn = attn_ref[...]
    node_msg = jnp.sum(mnh_ref[...] * attn[:, :, None], axis=1)
    edge_msg = jnp.sum(meh_ref[...], axis=1)
    h = (jnp.dot(edge_msg, w1_ref[0 * D:1 * D], preferred_element_type=jnp.float32)
         + jnp.dot(node_msg, w1_ref[1 * D:2 * D], preferred_element_type=jnp.float32)
         + jnp.dot(nh_ref[...], w1_ref[2 * D:3 * D], preferred_element_type=jnp.float32)
         + jnp.dot(nf_ref[...], w1_ref[3 * D:4 * D], preferred_element_type=jnp.float32)
         + b1_ref[...])
    h = jnp.maximum(h, 0.0)
    h = jnp.dot(h, w2_ref[...], preferred_element_type=jnp.float32) + b2_ref[...]
    h = jnp.maximum(h, 0.0)
    out_ref[...] = jnp.dot(h, w3_ref[...], preferred_element_type=jnp.float32) + b3_ref[...]


def kernel(mailbox_node_h, mailbox_attn, mailbox_edge_h, node_h, node_features,
           W1, b1, W2, b2, W3, b3):
    attn2d = mailbox_attn[:, :, 0]
    b1r = b1.reshape(1, HIDDEN)
    b2r = b2.reshape(1, HIDDEN)
    b3r = b3.reshape(1, OUT)

    grid = (N // BN,)
    row = lambda i: (i, 0)
    row3 = lambda i: (i, 0, 0)
    fixed = lambda i: (0, 0)

    return pl.pallas_call(
        _fused,
        grid=grid,
        in_specs=[
            pl.BlockSpec((BN, DEG, D), row3),
            pl.BlockSpec((BN, DEG), row),
            pl.BlockSpec((BN, DEG, D), row3),
            pl.BlockSpec((BN, D), row),
            pl.BlockSpec((BN, D), row),
            pl.BlockSpec((4 * D, HIDDEN), fixed),
            pl.BlockSpec((1, HIDDEN), fixed),
            pl.BlockSpec((HIDDEN, HIDDEN), fixed),
            pl.BlockSpec((1, HIDDEN), fixed),
            pl.BlockSpec((HIDDEN, OUT), fixed),
            pl.BlockSpec((1, OUT), fixed),
        ],
        out_specs=pl.BlockSpec((BN, OUT), row),
        out_shape=jax.ShapeDtypeStruct((N, OUT), jnp.float32),
        compiler_params=pltpu.CompilerParams(
            dimension_semantics=("arbitrary",),
        ),
    )(mailbox_node_h, attn2d, mailbox_edge_h, node_h, node_features,
      W1, b1r, W2, b2r, W3, b3r)
